# baseline (speedup 1.0000x reference)
import jax
import jax.numpy as jnp
from jax import lax
from jax.experimental import pallas as pl
from jax.experimental.pallas import tpu as pltpu
from jax.experimental.pallas import tpu_sc as plsc

_BATCH = 16384
_FEAT = 64
_CLASSES = 100000
_NW = 32
_FPW = _FEAT // _NW
_QTR = _BATCH // 4


def _body(embT_hbm, lab_hbm, cenT_hbm, out_hbm,
          row_v, lab_v, emb_v, acc_v, lab_sh, sem, esem):
    cid = lax.axis_index("c")
    sid = lax.axis_index("s")
    w = sid * 2 + cid

    pltpu.async_copy(cenT_hbm.at[w * _FPW], row_v, sem)
    pltpu.async_copy(embT_hbm.at[w * _FPW, pl.ds(0, _QTR)], emb_v.at[0], esem)

    @pl.when(sid == 0)
    def _():
        pltpu.sync_copy(lab_hbm, lab_sh)

    plsc.subcore_barrier()
    pltpu.sync_copy(lab_sh, lab_v)

    zero = jnp.zeros((16,), jnp.float32)
    accs = (zero,) * 8

    def feat_body(k, accs):
        f = w * _FPW + k

        @pl.when(k > 0)
        def _():
            pltpu.async_copy(embT_hbm.at[f, pl.ds(0, _QTR)], emb_v.at[0], esem)
            pltpu.async_copy(cenT_hbm.at[f], row_v, sem)

        pltpu.make_async_copy(cenT_hbm.at[f], row_v, sem).wait()

        def qtr_body(q, accs):
            qmod = lax.rem(q, 2)

            @pl.when(q < 3)
            def _():
                pltpu.async_copy(
                    embT_hbm.at[f, pl.ds((q + 1) * _QTR, _QTR)],
                    emb_v.at[1 - qmod], esem)

            pltpu.make_async_copy(
                embT_hbm.at[f, pl.ds(0, _QTR)], emb_v.at[0], esem).wait()

            @plsc.parallel_loop(0, _QTR, step=128, carry=accs)
            def accs(t, accs):
                out = []
                for u in range(8):
                    idx16 = lab_v[pl.ds(q * _QTR + t + u * 16, 16)]
                    g = plsc.load_gather(row_v, [idx16])
                    e = emb_v[qmod, pl.ds(t + u * 16, 16)]
                    d = e - g
                    out.append(accs[u] + d * d)
                return tuple(out)

            return accs

        return lax.fori_loop(0, 4, qtr_body, accs)

    accs = lax.fori_loop(0, _FPW, feat_body, accs)

    acc = ((accs[0] + accs[1]) + (accs[2] + accs[3])
           + (accs[4] + accs[5]) + (accs[6] + accs[7]))
    acc_v[...] = acc * (1.0 / (2.0 * _BATCH))
    pltpu.sync_copy(acc_v, out_hbm.at[w])


@jax.jit
def _center_loss(embeddings, labels, centers):
    lab = labels.astype(jnp.int32)
    embT = embeddings.T
    cenT = centers.T
    kern = pl.kernel(
        _body,
        out_type=jax.ShapeDtypeStruct((_NW, 16), jnp.float32),
        mesh=plsc.VectorSubcoreMesh(core_axis_name="c", subcore_axis_name="s"),
        scratch_types=[
            pltpu.VMEM((_CLASSES,), jnp.float32),
            pltpu.VMEM((_BATCH,), jnp.int32),
            pltpu.VMEM((2, _QTR), jnp.float32),
            pltpu.VMEM((16,), jnp.float32),
            pltpu.VMEM_SHARED((_BATCH,), jnp.int32),
            pltpu.SemaphoreType.DMA,
            pltpu.SemaphoreType.DMA,
        ],
        compiler_params=pltpu.CompilerParams(needs_layout_passes=False),
    )
    partials = kern(embT, lab, cenT)
    return jnp.sum(partials)


def kernel(embeddings, labels, centers):
    return _center_loss(embeddings, labels, centers)
